# baseline (speedup 1.0000x reference)
import functools

import jax
import jax.numpy as jnp
from jax import lax
from jax.experimental import pallas as pl
from jax.experimental.pallas import tpu as pltpu
from jax.experimental.pallas import tpu_sc as plsc

N = 10000
E = 320000
D = 128
C = 10
L = 4
EPS = 1e-5

NC = 2
NS = 16
K = 128
NCHUNK = E // K
CPT = 80
NCHUNK_PAD = NC * NS * CPT
E_PAD = NCHUNK_PAD * K
N_ACC = N + 8
ROWS_MAIN = 624
ROWS_LAST = N - (NS - 1) * ROWS_MAIN
NPAD = 10240

_mesh = plsc.VectorSubcoreMesh(core_axis_name="c", subcore_axis_name="s")


@functools.partial(
    pl.kernel,
    out_type=jax.ShapeDtypeStruct((NC * NPAD,), jnp.float32),
    mesh=_mesh,
    scratch_types=[
        pltpu.VMEM((CPT, K), jnp.int32),
        pltpu.VMEM((K,), jnp.float32),
        pltpu.VMEM_SHARED((NPAD,), jnp.float32),
    ],
)
def _sc_degree(dst_hbm, zeros_hbm, out_hbm, didx_all, ones_v, acc):
    c = lax.axis_index("c")
    s = lax.axis_index("s")
    w = c * NS + s

    @pl.when(s == 0)
    def _():
        pltpu.sync_copy(zeros_hbm, acc)

    for i in range(K // 16):
        ones_v[pl.ds(i * 16, 16)] = jnp.ones((16,), jnp.float32)

    pltpu.sync_copy(dst_hbm.at[pl.ds(pl.multiple_of(w * CPT, 8), CPT)], didx_all)
    plsc.subcore_barrier()

    @pl.loop(0, CPT)
    def _(j):
        pltpu.sync_copy(ones_v, acc.at[didx_all.at[j]], add=True)

    plsc.subcore_barrier()

    @pl.when(s == 0)
    def _():
        pltpu.sync_copy(acc, out_hbm.at[pl.ds(pl.multiple_of(c * NPAD, 8), NPAD)])


@functools.partial(
    pl.kernel,
    out_type=jax.ShapeDtypeStruct((NC, N, D), jnp.float32),
    mesh=_mesh,
    scratch_types=[
        pltpu.VMEM((CPT, K), jnp.int32),
        pltpu.VMEM((K,), jnp.int32),
        pltpu.VMEM((K,), jnp.int32),
        pltpu.VMEM((K, D), jnp.float32),
        pltpu.VMEM((K, D), jnp.float32),
        pltpu.VMEM_SHARED((N_ACC, D), jnp.float32),
        pltpu.SemaphoreType.DMA,
        pltpu.SemaphoreType.DMA,
        pltpu.SemaphoreType.DMA,
        pltpu.SemaphoreType.DMA,
    ],
)
def _sc_gather_scatter(g_hbm, src_hbm, dst_hbm, zeros_hbm, out_hbm,
                       sidx_all, didx0, didx1, rows0, rows1, acc,
                       gsem0, gsem1, dsem0, dsem1):
    c = lax.axis_index("c")
    s = lax.axis_index("s")
    w = c * NS + s
    row0 = pl.multiple_of(s * ROWS_MAIN, 8)
    blk = pl.multiple_of(w * CPT, 8)
    ch0 = w * CPT

    pltpu.sync_copy(src_hbm.at[pl.ds(blk, CPT)], sidx_all)

    @pl.when(s < NS - 1)
    def _():
        pltpu.sync_copy(zeros_hbm.at[pl.ds(row0, ROWS_MAIN)],
                        acc.at[pl.ds(row0, ROWS_MAIN)])

    @pl.when(s == NS - 1)
    def _():
        pltpu.sync_copy(zeros_hbm.at[pl.ds((NS - 1) * ROWS_MAIN, ROWS_LAST)],
                        acc.at[pl.ds((NS - 1) * ROWS_MAIN, ROWS_LAST)])

    plsc.subcore_barrier()

    @pl.loop(0, CPT)
    def _(j):
        pltpu.sync_copy(src_hbm.at[ch0 + j], didx1)
        pltpu.sync_copy(dst_hbm.at[ch0 + j], didx0)
        pltpu.async_copy(g_hbm.at[didx1], rows0, gsem0).wait()
        pltpu.sync_copy(rows0, acc.at[didx0], add=True)

    plsc.subcore_barrier()

    @pl.when(s < NS - 1)
    def _():
        pltpu.sync_copy(acc.at[pl.ds(row0, ROWS_MAIN)],
                        out_hbm.at[c, pl.ds(row0, ROWS_MAIN)])

    @pl.when(s == NS - 1)
    def _():
        pltpu.sync_copy(acc.at[pl.ds((NS - 1) * ROWS_MAIN, ROWS_LAST)],
                        out_hbm.at[c, pl.ds((NS - 1) * ROWS_MAIN, ROWS_LAST)])


def _tc_pre_body(d0_ref, d1_ref, h_ref, norm_ref, g_ref):
    deg = jnp.maximum(d0_ref[...] + d1_ref[...], 1.0)
    norm = lax.rsqrt(deg)
    norm_ref[...] = norm
    g_ref[...] = h_ref[...] * norm


def _tc_pre(d0, d1, h):
    return pl.pallas_call(
        _tc_pre_body,
        out_shape=[
            jax.ShapeDtypeStruct((N, 1), jnp.float32),
            jax.ShapeDtypeStruct((N, D), jnp.float32),
        ],
    )(d0, d1, h)


def _batchnorm_relu(x, gamma, beta):
    mean = jnp.mean(x, axis=0, keepdims=True)
    xc = x - mean
    var = jnp.mean(xc * xc, axis=0, keepdims=True)
    return jnp.maximum(xc * lax.rsqrt(var + EPS) * gamma + beta, 0.0)


def _tc_layer_body(a0_ref, a1_ref, norm_ref, gamma_ref, beta_ref, g_ref):
    x = (a0_ref[...] + a1_ref[...]) * norm_ref[...]
    y = _batchnorm_relu(x, gamma_ref[...], beta_ref[...])
    g_ref[...] = y * norm_ref[...]


def _tc_layer(a0, a1, norm, gamma, beta):
    return pl.pallas_call(
        _tc_layer_body,
        out_shape=jax.ShapeDtypeStruct((N, D), jnp.float32),
    )(a0, a1, norm, gamma, beta)


def _tc_final_body(a0_ref, a1_ref, norm_ref, gamma_ref, beta_ref,
                   W1_ref, b1_ref, mg_ref, mb_ref, W2_ref, b2_ref, out_ref):
    x = (a0_ref[...] + a1_ref[...]) * norm_ref[...]
    y = _batchnorm_relu(x, gamma_ref[...], beta_ref[...])
    x1 = jnp.dot(y, W1_ref[...], preferred_element_type=jnp.float32) + b1_ref[...]
    m1 = jnp.mean(x1, axis=0, keepdims=True)
    x1c = x1 - m1
    v1 = jnp.mean(x1c * x1c, axis=0, keepdims=True)
    xn = x1c * lax.rsqrt(v1 + EPS) * mg_ref[...] + mb_ref[...]
    out_ref[...] = (jnp.dot(xn, W2_ref[...], preferred_element_type=jnp.float32)
                    + b2_ref[...])


def _tc_final(a0, a1, norm, gamma, beta, W1, b1, mg, mb, W2, b2):
    return pl.pallas_call(
        _tc_final_body,
        out_shape=jax.ShapeDtypeStruct((N, C), jnp.float32),
    )(a0, a1, norm, gamma, beta, W1, b1, mg, mb, W2, b2)


def kernel(h, edge_index, e, bn_gamma, bn_beta, W1, b1,
           mbn_gamma, mbn_beta, W2, b2):
    del e
    src = edge_index[0]
    dst = edge_index[1]

    pad = E_PAD - E
    src2d = jnp.concatenate(
        [src, jnp.zeros((pad,), jnp.int32)]).reshape(NCHUNK_PAD, K)
    dst2d = jnp.concatenate(
        [dst, jnp.full((pad,), N, jnp.int32)]).reshape(NCHUNK_PAD, K)

    zeros_n = jnp.zeros((NPAD,), jnp.float32)
    zeros_nd = jnp.zeros((N, D), jnp.float32)

    deg2 = _sc_degree(dst2d, zeros_n)
    norm, g = _tc_pre(deg2[:N].reshape(N, 1),
                      deg2[NPAD:NPAD + N].reshape(N, 1), h)

    gamma2 = bn_gamma.reshape(1, D)
    beta2 = bn_beta.reshape(1, D)

    out = None
    for layer in range(L):
        agg = _sc_gather_scatter(g, src2d, dst2d, zeros_nd)
        if layer < L - 1:
            g = _tc_layer(agg[0], agg[1], norm, gamma2, beta2)
        else:
            out = _tc_final(agg[0], agg[1], norm, gamma2, beta2,
                            W1, b1.reshape(1, D),
                            mbn_gamma.reshape(1, D), mbn_beta.reshape(1, D),
                            W2, b2.reshape(1, C))
    return out


# speedup vs baseline: 1.0015x; 1.0015x over previous
import functools

import jax
import jax.numpy as jnp
from jax import lax
from jax.experimental import pallas as pl
from jax.experimental.pallas import tpu as pltpu
from jax.experimental.pallas import tpu_sc as plsc

N = 10000
E = 320000
D = 128
C = 10
L = 4
EPS = 1e-5

NC = 2
NS = 16
K = 128
NCHUNK = E // K
CPT = 80
NCHUNK_PAD = NC * NS * CPT
E_PAD = NCHUNK_PAD * K
N_ACC = N + K
ROWS_MAIN = 624
ROWS_LAST = N - (NS - 1) * ROWS_MAIN
NPAD = 10240

_mesh = plsc.VectorSubcoreMesh(core_axis_name="c", subcore_axis_name="s")


@functools.partial(
    pl.kernel,
    out_type=jax.ShapeDtypeStruct((NC * NPAD,), jnp.float32),
    mesh=_mesh,
    scratch_types=[
        pltpu.VMEM((CPT, K), jnp.int32),
        pltpu.VMEM((K,), jnp.float32),
        pltpu.VMEM_SHARED((NPAD,), jnp.float32),
    ],
)
def _sc_degree(dst_hbm, zeros_hbm, out_hbm, didx_all, ones_v, acc):
    c = lax.axis_index("c")
    s = lax.axis_index("s")
    w = c * NS + s

    @pl.when(s == 0)
    def _():
        pltpu.sync_copy(zeros_hbm, acc)

    for i in range(K // 16):
        ones_v[pl.ds(i * 16, 16)] = jnp.ones((16,), jnp.float32)

    pltpu.sync_copy(dst_hbm.at[pl.ds(pl.multiple_of(w * CPT, 8), CPT)], didx_all)
    plsc.subcore_barrier()

    @pl.loop(0, CPT)
    def _(j):
        pltpu.sync_copy(ones_v, acc.at[didx_all.at[j]], add=True)

    plsc.subcore_barrier()

    @pl.when(s == 0)
    def _():
        pltpu.sync_copy(acc, out_hbm.at[pl.ds(pl.multiple_of(c * NPAD, 8), NPAD)])


@functools.partial(
    pl.kernel,
    out_type=jax.ShapeDtypeStruct((NC, N, D), jnp.float32),
    mesh=_mesh,
    scratch_types=[
        pltpu.VMEM((CPT, K), jnp.int32),
        pltpu.VMEM((K,), jnp.int32),
        pltpu.VMEM((K,), jnp.int32),
        pltpu.VMEM((K, D), jnp.float32),
        pltpu.VMEM((K, D), jnp.float32),
        pltpu.VMEM_SHARED((N_ACC, D), jnp.float32),
        pltpu.SemaphoreType.DMA,
        pltpu.SemaphoreType.DMA,
        pltpu.SemaphoreType.DMA,
        pltpu.SemaphoreType.DMA,
    ],
)
def _sc_gather_scatter(g_hbm, src_hbm, dst_hbm, zeros_hbm, out_hbm,
                       sidx_all, didx0, didx1, rows0, rows1, acc,
                       gsem0, gsem1, dsem0, dsem1):
    c = lax.axis_index("c")
    s = lax.axis_index("s")
    w = c * NS + s
    row0 = pl.multiple_of(s * ROWS_MAIN, 8)
    blk = pl.multiple_of(w * CPT, 8)
    ch0 = w * CPT

    pltpu.sync_copy(src_hbm.at[pl.ds(blk, CPT)], sidx_all)

    @pl.when(s < NS - 1)
    def _():
        pltpu.sync_copy(zeros_hbm.at[pl.ds(row0, ROWS_MAIN)],
                        acc.at[pl.ds(row0, ROWS_MAIN)])

    @pl.when(s == NS - 1)
    def _():
        pltpu.sync_copy(zeros_hbm.at[pl.ds((NS - 1) * ROWS_MAIN, ROWS_LAST)],
                        acc.at[pl.ds((NS - 1) * ROWS_MAIN, ROWS_LAST)])

    plsc.subcore_barrier()

    @pl.loop(0, CPT)
    def _(j):
        pltpu.sync_copy(src_hbm.at[ch0 + j], didx1)
        pltpu.sync_copy(dst_hbm.at[ch0 + j], didx0)
        pltpu.async_copy(g_hbm.at[didx1], rows0, gsem0).wait()
        pltpu.sync_copy(rows0, acc.at[didx0], add=True)

    plsc.subcore_barrier()

    @pl.when(s < NS - 1)
    def _():
        pltpu.sync_copy(acc.at[pl.ds(row0, ROWS_MAIN)],
                        out_hbm.at[c, pl.ds(row0, ROWS_MAIN)])

    @pl.when(s == NS - 1)
    def _():
        pltpu.sync_copy(acc.at[pl.ds((NS - 1) * ROWS_MAIN, ROWS_LAST)],
                        out_hbm.at[c, pl.ds((NS - 1) * ROWS_MAIN, ROWS_LAST)])


def _tc_pre_body(d0_ref, d1_ref, h_ref, norm_ref, g_ref):
    deg = jnp.maximum(d0_ref[...] + d1_ref[...], 1.0)
    norm = lax.rsqrt(deg)
    norm_ref[...] = norm
    g_ref[...] = h_ref[...] * norm


def _tc_pre(d0, d1, h):
    return pl.pallas_call(
        _tc_pre_body,
        out_shape=[
            jax.ShapeDtypeStruct((N, 1), jnp.float32),
            jax.ShapeDtypeStruct((N, D), jnp.float32),
        ],
    )(d0, d1, h)


def _batchnorm_relu(x, gamma, beta):
    mean = jnp.mean(x, axis=0, keepdims=True)
    xc = x - mean
    var = jnp.mean(xc * xc, axis=0, keepdims=True)
    return jnp.maximum(xc * lax.rsqrt(var + EPS) * gamma + beta, 0.0)


def _tc_layer_body(a0_ref, a1_ref, norm_ref, gamma_ref, beta_ref, g_ref):
    x = (a0_ref[...] + a1_ref[...]) * norm_ref[...]
    y = _batchnorm_relu(x, gamma_ref[...], beta_ref[...])
    g_ref[...] = y * norm_ref[...]


def _tc_layer(a0, a1, norm, gamma, beta):
    return pl.pallas_call(
        _tc_layer_body,
        out_shape=jax.ShapeDtypeStruct((N, D), jnp.float32),
    )(a0, a1, norm, gamma, beta)


def _tc_final_body(a0_ref, a1_ref, norm_ref, gamma_ref, beta_ref,
                   W1_ref, b1_ref, mg_ref, mb_ref, W2_ref, b2_ref, out_ref):
    x = (a0_ref[...] + a1_ref[...]) * norm_ref[...]
    y = _batchnorm_relu(x, gamma_ref[...], beta_ref[...])
    x1 = jnp.dot(y, W1_ref[...], preferred_element_type=jnp.float32) + b1_ref[...]
    m1 = jnp.mean(x1, axis=0, keepdims=True)
    x1c = x1 - m1
    v1 = jnp.mean(x1c * x1c, axis=0, keepdims=True)
    xn = x1c * lax.rsqrt(v1 + EPS) * mg_ref[...] + mb_ref[...]
    out_ref[...] = (jnp.dot(xn, W2_ref[...], preferred_element_type=jnp.float32)
                    + b2_ref[...])


def _tc_final(a0, a1, norm, gamma, beta, W1, b1, mg, mb, W2, b2):
    return pl.pallas_call(
        _tc_final_body,
        out_shape=jax.ShapeDtypeStruct((N, C), jnp.float32),
    )(a0, a1, norm, gamma, beta, W1, b1, mg, mb, W2, b2)


def kernel(h, edge_index, e, bn_gamma, bn_beta, W1, b1,
           mbn_gamma, mbn_beta, W2, b2):
    del e
    src = edge_index[0]
    dst = edge_index[1]

    pad = E_PAD - E
    src2d = jnp.concatenate(
        [src, jnp.zeros((pad,), jnp.int32)]).reshape(NCHUNK_PAD, K)
    dst2d = jnp.concatenate(
        [dst, N + (jnp.arange(pad, dtype=jnp.int32) % K)]).reshape(NCHUNK_PAD, K)

    zeros_n = jnp.zeros((NPAD,), jnp.float32)
    zeros_nd = jnp.zeros((N, D), jnp.float32)

    deg2 = _sc_degree(dst2d, zeros_n)
    norm, g = _tc_pre(deg2[:N].reshape(N, 1),
                      deg2[NPAD:NPAD + N].reshape(N, 1), h)

    gamma2 = bn_gamma.reshape(1, D)
    beta2 = bn_beta.reshape(1, D)

    out = None
    for layer in range(L):
        agg = _sc_gather_scatter(g, src2d, dst2d, zeros_nd)
        if layer < L - 1:
            g = _tc_layer(agg[0], agg[1], norm, gamma2, beta2)
        else:
            out = _tc_final(agg[0], agg[1], norm, gamma2, beta2,
                            W1, b1.reshape(1, D),
                            mbn_gamma.reshape(1, D), mbn_beta.reshape(1, D),
                            W2, b2.reshape(1, C))
    return out


# speedup vs baseline: 1.3343x; 1.3324x over previous
import functools

import jax
import jax.numpy as jnp
from jax import lax
from jax.experimental import pallas as pl
from jax.experimental.pallas import tpu as pltpu
from jax.experimental.pallas import tpu_sc as plsc

N = 10000
E = 320000
D = 128
C = 10
L = 4
EPS = 1e-5

NC = 2
NS = 16
K = 128
NCHUNK = E // K
CPT = 80
NCHUNK_PAD = NC * NS * CPT
E_PAD = NCHUNK_PAD * K
N_ACC = N + K
ROWS_MAIN = 624
ROWS_LAST = N - (NS - 1) * ROWS_MAIN
NPAD = 10240

_mesh = plsc.VectorSubcoreMesh(core_axis_name="c", subcore_axis_name="s")


@functools.partial(
    pl.kernel,
    out_type=jax.ShapeDtypeStruct((NC * NPAD,), jnp.float32),
    mesh=_mesh,
    scratch_types=[
        pltpu.VMEM((CPT, K), jnp.int32),
        pltpu.VMEM((K,), jnp.float32),
        pltpu.VMEM_SHARED((NPAD,), jnp.float32),
    ],
)
def _sc_degree(dst_hbm, zeros_hbm, out_hbm, didx_all, ones_v, acc):
    c = lax.axis_index("c")
    s = lax.axis_index("s")
    w = c * NS + s

    @pl.when(s == 0)
    def _():
        pltpu.sync_copy(zeros_hbm, acc)

    for i in range(K // 16):
        ones_v[pl.ds(i * 16, 16)] = jnp.ones((16,), jnp.float32)

    pltpu.sync_copy(dst_hbm.at[pl.ds(pl.multiple_of(w * CPT, 8), CPT)], didx_all)
    plsc.subcore_barrier()

    @pl.loop(0, CPT)
    def _(j):
        pltpu.sync_copy(ones_v, acc.at[didx_all.at[j]], add=True)

    plsc.subcore_barrier()

    @pl.when(s == 0)
    def _():
        pltpu.sync_copy(acc, out_hbm.at[pl.ds(pl.multiple_of(c * NPAD, 8), NPAD)])


@functools.partial(
    pl.kernel,
    out_type=jax.ShapeDtypeStruct((NC, N, D), jnp.float32),
    mesh=_mesh,
    scratch_types=[
        pltpu.VMEM((K,), jnp.int32),
        pltpu.VMEM((K,), jnp.int32),
        pltpu.VMEM((K,), jnp.int32),
        pltpu.VMEM((K,), jnp.int32),
        pltpu.VMEM((K, D), jnp.float32),
        pltpu.VMEM((K, D), jnp.float32),
        pltpu.VMEM_SHARED((N_ACC, D), jnp.float32),
        pltpu.SemaphoreType.DMA,
        pltpu.SemaphoreType.DMA,
    ],
)
def _sc_gather_scatter(g_hbm, src_hbm, dst_hbm, zeros_hbm, out_hbm,
                       sidx0, sidx1, didx0, didx1, rows0, rows1, acc,
                       gsem0, gsem1):
    c = lax.axis_index("c")
    s = lax.axis_index("s")
    w = c * NS + s
    row0 = pl.multiple_of(s * ROWS_MAIN, 8)

    @pl.when(s < NS - 1)
    def _():
        pltpu.sync_copy(zeros_hbm.at[pl.ds(row0, ROWS_MAIN)],
                        acc.at[pl.ds(row0, ROWS_MAIN)])

    @pl.when(s == NS - 1)
    def _():
        pltpu.sync_copy(zeros_hbm.at[pl.ds((NS - 1) * ROWS_MAIN, ROWS_LAST)],
                        acc.at[pl.ds((NS - 1) * ROWS_MAIN, ROWS_LAST)])

    plsc.subcore_barrier()

    @pl.loop(0, CPT, step=2)
    def _(j):
        base_a = pl.multiple_of((w + j * NC * NS) * K, 8)
        base_b = pl.multiple_of((w + (j + 1) * NC * NS) * K, 8)
        pltpu.sync_copy(src_hbm.at[pl.ds(base_a, K)], sidx0)
        cp0 = pltpu.async_copy(g_hbm.at[sidx0], rows0, gsem0)
        pltpu.sync_copy(src_hbm.at[pl.ds(base_b, K)], sidx1)
        cp1 = pltpu.async_copy(g_hbm.at[sidx1], rows1, gsem1)
        pltpu.sync_copy(dst_hbm.at[pl.ds(base_a, K)], didx0)
        cp0.wait()
        pltpu.sync_copy(rows0, acc.at[didx0], add=True)
        pltpu.sync_copy(dst_hbm.at[pl.ds(base_b, K)], didx1)
        cp1.wait()
        pltpu.sync_copy(rows1, acc.at[didx1], add=True)

    plsc.subcore_barrier()

    @pl.when(s < NS - 1)
    def _():
        pltpu.sync_copy(acc.at[pl.ds(row0, ROWS_MAIN)],
                        out_hbm.at[c, pl.ds(row0, ROWS_MAIN)])

    @pl.when(s == NS - 1)
    def _():
        pltpu.sync_copy(acc.at[pl.ds((NS - 1) * ROWS_MAIN, ROWS_LAST)],
                        out_hbm.at[c, pl.ds((NS - 1) * ROWS_MAIN, ROWS_LAST)])


def _tc_pre_body(d0_ref, d1_ref, h_ref, norm_ref, g_ref):
    deg = jnp.maximum(d0_ref[...] + d1_ref[...], 1.0)
    norm = lax.rsqrt(deg)
    norm_ref[...] = norm
    g_ref[...] = h_ref[...] * norm


def _tc_pre(d0, d1, h):
    return pl.pallas_call(
        _tc_pre_body,
        out_shape=[
            jax.ShapeDtypeStruct((N, 1), jnp.float32),
            jax.ShapeDtypeStruct((N, D), jnp.float32),
        ],
    )(d0, d1, h)


def _batchnorm_relu(x, gamma, beta):
    mean = jnp.mean(x, axis=0, keepdims=True)
    xc = x - mean
    var = jnp.mean(xc * xc, axis=0, keepdims=True)
    return jnp.maximum(xc * lax.rsqrt(var + EPS) * gamma + beta, 0.0)


def _tc_layer_body(a0_ref, a1_ref, norm_ref, gamma_ref, beta_ref, g_ref):
    x = (a0_ref[...] + a1_ref[...]) * norm_ref[...]
    y = _batchnorm_relu(x, gamma_ref[...], beta_ref[...])
    g_ref[...] = y * norm_ref[...]


def _tc_layer(a0, a1, norm, gamma, beta):
    return pl.pallas_call(
        _tc_layer_body,
        out_shape=jax.ShapeDtypeStruct((N, D), jnp.float32),
    )(a0, a1, norm, gamma, beta)


def _tc_final_body(a0_ref, a1_ref, norm_ref, gamma_ref, beta_ref,
                   W1_ref, b1_ref, mg_ref, mb_ref, W2_ref, b2_ref, out_ref):
    x = (a0_ref[...] + a1_ref[...]) * norm_ref[...]
    y = _batchnorm_relu(x, gamma_ref[...], beta_ref[...])
    x1 = jnp.dot(y, W1_ref[...], preferred_element_type=jnp.float32) + b1_ref[...]
    m1 = jnp.mean(x1, axis=0, keepdims=True)
    x1c = x1 - m1
    v1 = jnp.mean(x1c * x1c, axis=0, keepdims=True)
    xn = x1c * lax.rsqrt(v1 + EPS) * mg_ref[...] + mb_ref[...]
    out_ref[...] = (jnp.dot(xn, W2_ref[...], preferred_element_type=jnp.float32)
                    + b2_ref[...])


def _tc_final(a0, a1, norm, gamma, beta, W1, b1, mg, mb, W2, b2):
    return pl.pallas_call(
        _tc_final_body,
        out_shape=jax.ShapeDtypeStruct((N, C), jnp.float32),
    )(a0, a1, norm, gamma, beta, W1, b1, mg, mb, W2, b2)


def kernel(h, edge_index, e, bn_gamma, bn_beta, W1, b1,
           mbn_gamma, mbn_beta, W2, b2):
    del e
    src = edge_index[0]
    dst = edge_index[1]

    pad = E_PAD - E
    src1d = jnp.concatenate([src, jnp.zeros((pad,), jnp.int32)])
    dst1d = jnp.concatenate(
        [dst, N + (jnp.arange(pad, dtype=jnp.int32) % K)])
    dst2d = dst1d.reshape(NCHUNK_PAD, K)

    zeros_n = jnp.zeros((NPAD,), jnp.float32)
    zeros_nd = jnp.zeros((N, D), jnp.float32)

    deg2 = _sc_degree(dst2d, zeros_n)
    norm, g = _tc_pre(deg2[:N].reshape(N, 1),
                      deg2[NPAD:NPAD + N].reshape(N, 1), h)

    gamma2 = bn_gamma.reshape(1, D)
    beta2 = bn_beta.reshape(1, D)

    out = None
    for layer in range(L):
        agg = _sc_gather_scatter(g, src1d, dst1d, zeros_nd)
        if layer < L - 1:
            g = _tc_layer(agg[0], agg[1], norm, gamma2, beta2)
        else:
            out = _tc_final(agg[0], agg[1], norm, gamma2, beta2,
                            W1, b1.reshape(1, D),
                            mbn_gamma.reshape(1, D), mbn_beta.reshape(1, D),
                            W2, b2.reshape(1, C))
    return out


# speedup vs baseline: 2.3329x; 1.7484x over previous
import functools

import jax
import jax.numpy as jnp
from jax import lax
from jax.experimental import pallas as pl
from jax.experimental.pallas import tpu as pltpu
from jax.experimental.pallas import tpu_sc as plsc

N = 10000
E = 320000
D = 128
C = 10
L = 4
EPS = 1e-5

NC = 2
NS = 16
K = 128
NCHUNK = E // K
CPT = 80
NCHUNK_PAD = NC * NS * CPT
E_PAD = NCHUNK_PAD * K
N_ACC = N + K
ROWS_MAIN = 624
ROWS_LAST = N - (NS - 1) * ROWS_MAIN
NPAD = 10240

_mesh = plsc.VectorSubcoreMesh(core_axis_name="c", subcore_axis_name="s")


@functools.partial(
    pl.kernel,
    out_type=jax.ShapeDtypeStruct((NC * NPAD,), jnp.float32),
    mesh=_mesh,
    scratch_types=[
        pltpu.VMEM((CPT, K), jnp.int32),
        pltpu.VMEM((K,), jnp.float32),
        pltpu.VMEM_SHARED((NPAD,), jnp.float32),
    ],
)
def _sc_degree(dst_hbm, zeros_hbm, out_hbm, didx_all, ones_v, acc):
    c = lax.axis_index("c")
    s = lax.axis_index("s")
    w = c * NS + s

    @pl.when(s == 0)
    def _():
        pltpu.sync_copy(zeros_hbm, acc)

    for i in range(K // 16):
        ones_v[pl.ds(i * 16, 16)] = jnp.ones((16,), jnp.float32)

    pltpu.sync_copy(dst_hbm.at[pl.ds(pl.multiple_of(w * CPT, 8), CPT)], didx_all)
    plsc.subcore_barrier()

    @pl.loop(0, CPT)
    def _(j):
        pltpu.sync_copy(ones_v, acc.at[didx_all.at[j]], add=True)

    plsc.subcore_barrier()

    @pl.when(s == 0)
    def _():
        pltpu.sync_copy(acc, out_hbm.at[pl.ds(pl.multiple_of(c * NPAD, 8), NPAD)])


@functools.partial(
    pl.kernel,
    out_type=jax.ShapeDtypeStruct((NC, N, D), jnp.float32),
    mesh=_mesh,
    scratch_types=[
        pltpu.VMEM((K,), jnp.int32),
        pltpu.VMEM((K,), jnp.int32),
        pltpu.VMEM((K,), jnp.int32),
        pltpu.VMEM((K,), jnp.int32),
        pltpu.VMEM((K, D), jnp.float32),
        pltpu.VMEM((K, D), jnp.float32),
        pltpu.VMEM_SHARED((N_ACC, D), jnp.float32),
        pltpu.SemaphoreType.DMA,
        pltpu.SemaphoreType.DMA,
    ],
)
def _sc_gather_scatter(g_hbm, src_hbm, dst_hbm, zeros_hbm, out_hbm,
                       sidx0, sidx1, didx0, didx1, rows0, rows1, acc,
                       gsem0, gsem1):
    c = lax.axis_index("c")
    s = lax.axis_index("s")
    w = c * NS + s
    row0 = pl.multiple_of(s * ROWS_MAIN, 8)

    @pl.when(s < NS - 1)
    def _():
        pltpu.sync_copy(zeros_hbm.at[pl.ds(row0, ROWS_MAIN)],
                        acc.at[pl.ds(row0, ROWS_MAIN)])

    @pl.when(s == NS - 1)
    def _():
        pltpu.sync_copy(zeros_hbm.at[pl.ds((NS - 1) * ROWS_MAIN, ROWS_LAST)],
                        acc.at[pl.ds((NS - 1) * ROWS_MAIN, ROWS_LAST)])

    plsc.subcore_barrier()

    @pl.loop(c * NS + s, NCHUNK, step=NC * NS)
    def _(ch):
        base = ch * K
        pltpu.sync_copy(src_hbm.at[pl.ds(base, K)], sidx0)
        pltpu.sync_copy(dst_hbm.at[pl.ds(base, K)], didx0)
        pltpu.async_copy(g_hbm.at[sidx0], rows0, gsem0).wait()
        pltpu.sync_copy(rows0, acc.at[didx0], add=True)

    plsc.subcore_barrier()

    @pl.when(s < NS - 1)
    def _():
        pltpu.sync_copy(acc.at[pl.ds(row0, ROWS_MAIN)],
                        out_hbm.at[c, pl.ds(row0, ROWS_MAIN)])

    @pl.when(s == NS - 1)
    def _():
        pltpu.sync_copy(acc.at[pl.ds((NS - 1) * ROWS_MAIN, ROWS_LAST)],
                        out_hbm.at[c, pl.ds((NS - 1) * ROWS_MAIN, ROWS_LAST)])


def _tc_pre_body(d0_ref, d1_ref, h_ref, norm_ref, g_ref):
    deg = jnp.maximum(d0_ref[...] + d1_ref[...], 1.0)
    norm = lax.rsqrt(deg)
    norm_ref[...] = norm
    g_ref[...] = h_ref[...] * norm


def _tc_pre(d0, d1, h):
    return pl.pallas_call(
        _tc_pre_body,
        out_shape=[
            jax.ShapeDtypeStruct((N, 1), jnp.float32),
            jax.ShapeDtypeStruct((N, D), jnp.float32),
        ],
    )(d0, d1, h)


def _batchnorm_relu(x, gamma, beta):
    mean = jnp.mean(x, axis=0, keepdims=True)
    xc = x - mean
    var = jnp.mean(xc * xc, axis=0, keepdims=True)
    return jnp.maximum(xc * lax.rsqrt(var + EPS) * gamma + beta, 0.0)


def _tc_layer_body(a0_ref, a1_ref, norm_ref, gamma_ref, beta_ref, g_ref):
    x = (a0_ref[...] + a1_ref[...]) * norm_ref[...]
    y = _batchnorm_relu(x, gamma_ref[...], beta_ref[...])
    g_ref[...] = y * norm_ref[...]


def _tc_layer(a0, a1, norm, gamma, beta):
    return pl.pallas_call(
        _tc_layer_body,
        out_shape=jax.ShapeDtypeStruct((N, D), jnp.float32),
    )(a0, a1, norm, gamma, beta)


def _tc_final_body(a0_ref, a1_ref, norm_ref, gamma_ref, beta_ref,
                   W1_ref, b1_ref, mg_ref, mb_ref, W2_ref, b2_ref, out_ref):
    x = (a0_ref[...] + a1_ref[...]) * norm_ref[...]
    y = _batchnorm_relu(x, gamma_ref[...], beta_ref[...])
    x1 = jnp.dot(y, W1_ref[...], preferred_element_type=jnp.float32) + b1_ref[...]
    m1 = jnp.mean(x1, axis=0, keepdims=True)
    x1c = x1 - m1
    v1 = jnp.mean(x1c * x1c, axis=0, keepdims=True)
    xn = x1c * lax.rsqrt(v1 + EPS) * mg_ref[...] + mb_ref[...]
    out_ref[...] = (jnp.dot(xn, W2_ref[...], preferred_element_type=jnp.float32)
                    + b2_ref[...])


def _tc_final(a0, a1, norm, gamma, beta, W1, b1, mg, mb, W2, b2):
    return pl.pallas_call(
        _tc_final_body,
        out_shape=jax.ShapeDtypeStruct((N, C), jnp.float32),
    )(a0, a1, norm, gamma, beta, W1, b1, mg, mb, W2, b2)


def kernel(h, edge_index, e, bn_gamma, bn_beta, W1, b1,
           mbn_gamma, mbn_beta, W2, b2):
    del e
    src = edge_index[0]
    dst = edge_index[1]

    pad = E_PAD - E
    src1d = jnp.concatenate([src, jnp.zeros((pad,), jnp.int32)])
    dst1d = jnp.concatenate(
        [dst, N + (jnp.arange(pad, dtype=jnp.int32) % K)])
    dst2d = dst1d.reshape(NCHUNK_PAD, K)

    zeros_n = jnp.zeros((NPAD,), jnp.float32)
    zeros_nd = jnp.zeros((N, D), jnp.float32)

    deg2 = _sc_degree(dst2d, zeros_n)
    norm, g = _tc_pre(deg2[:N].reshape(N, 1),
                      deg2[NPAD:NPAD + N].reshape(N, 1), h)

    gamma2 = bn_gamma.reshape(1, D)
    beta2 = bn_beta.reshape(1, D)

    out = None
    for layer in range(L):
        agg = _sc_gather_scatter(g, src1d, dst1d, zeros_nd)
        if layer < L - 1:
            g = _tc_layer(agg[0], agg[1], norm, gamma2, beta2)
        else:
            out = _tc_final(agg[0], agg[1], norm, gamma2, beta2,
                            W1, b1.reshape(1, D),
                            mbn_gamma.reshape(1, D), mbn_beta.reshape(1, D),
                            W2, b2.reshape(1, C))
    return out


# speedup vs baseline: 3.1589x; 1.3541x over previous
import functools

import jax
import jax.numpy as jnp
from jax import lax
from jax.experimental import pallas as pl
from jax.experimental.pallas import tpu as pltpu
from jax.experimental.pallas import tpu_sc as plsc

N = 10000
E = 320000
D = 128
C = 10
L = 4
EPS = 1e-5

NC = 2
NS = 16
K = 128
NCHUNK = E // K
CPT = 80
NCHUNK_PAD = NC * NS * CPT
E_PAD = NCHUNK_PAD * K
N_ACC = N + K
ROWS_MAIN = 624
ROWS_LAST = N - (NS - 1) * ROWS_MAIN
NPAD = 10240

_mesh = plsc.VectorSubcoreMesh(core_axis_name="c", subcore_axis_name="s")


@functools.partial(
    pl.kernel,
    out_type=jax.ShapeDtypeStruct((NC * NPAD,), jnp.float32),
    mesh=_mesh,
    scratch_types=[
        pltpu.VMEM((CPT, K), jnp.int32),
        pltpu.VMEM((K,), jnp.float32),
        pltpu.VMEM_SHARED((NPAD,), jnp.float32),
    ],
)
def _sc_degree(dst_hbm, zeros_hbm, out_hbm, didx_all, ones_v, acc):
    c = lax.axis_index("c")
    s = lax.axis_index("s")
    w = c * NS + s

    @pl.when(s == 0)
    def _():
        pltpu.sync_copy(zeros_hbm, acc)

    for i in range(K // 16):
        ones_v[pl.ds(i * 16, 16)] = jnp.ones((16,), jnp.float32)

    pltpu.sync_copy(dst_hbm.at[pl.ds(pl.multiple_of(w * CPT, 8), CPT)], didx_all)
    plsc.subcore_barrier()

    @pl.loop(0, CPT)
    def _(j):
        pltpu.sync_copy(ones_v, acc.at[didx_all.at[j]], add=True)

    plsc.subcore_barrier()

    @pl.when(s == 0)
    def _():
        pltpu.sync_copy(acc, out_hbm.at[pl.ds(pl.multiple_of(c * NPAD, 8), NPAD)])


@functools.partial(
    pl.kernel,
    out_type=jax.ShapeDtypeStruct((NC, N, D), jnp.float32),
    mesh=_mesh,
    scratch_types=[
        pltpu.VMEM((K,), jnp.int32),
        pltpu.VMEM((K,), jnp.int32),
        pltpu.VMEM((K,), jnp.int32),
        pltpu.VMEM((K,), jnp.int32),
        pltpu.VMEM((K, D), jnp.float32),
        pltpu.VMEM((K, D), jnp.float32),
        pltpu.VMEM_SHARED((N_ACC, D), jnp.float32),
        pltpu.SemaphoreType.DMA,
        pltpu.SemaphoreType.DMA,
    ],
)
def _sc_gather_scatter(g_hbm, src_hbm, dst_hbm, zeros_hbm, out_hbm,
                       sidx0, sidx1, didx0, didx1, rows0, rows1, acc,
                       gsem0, gsem1):
    c = lax.axis_index("c")
    s = lax.axis_index("s")
    w = c * NS + s
    row0 = pl.multiple_of(s * ROWS_MAIN, 8)

    @pl.when(s < NS - 1)
    def _():
        pltpu.sync_copy(zeros_hbm.at[pl.ds(row0, ROWS_MAIN)],
                        acc.at[pl.ds(row0, ROWS_MAIN)])

    @pl.when(s == NS - 1)
    def _():
        pltpu.sync_copy(zeros_hbm.at[pl.ds((NS - 1) * ROWS_MAIN, ROWS_LAST)],
                        acc.at[pl.ds((NS - 1) * ROWS_MAIN, ROWS_LAST)])

    plsc.subcore_barrier()

    @pl.loop(c * NS + s, NCHUNK, step=2 * NC * NS)
    def _(ch):
        base0 = ch * K
        pltpu.sync_copy(src_hbm.at[pl.ds(base0, K)], sidx0)
        cp0 = pltpu.async_copy(g_hbm.at[sidx0], rows0, gsem0)
        have_b = ch + NC * NS < NCHUNK

        @pl.when(have_b)
        def _():
            base1 = (ch + NC * NS) * K
            pltpu.sync_copy(src_hbm.at[pl.ds(base1, K)], sidx1)
            pltpu.async_copy(g_hbm.at[sidx1], rows1, gsem1)

        pltpu.sync_copy(dst_hbm.at[pl.ds(base0, K)], didx0)
        cp0.wait()
        pltpu.sync_copy(rows0, acc.at[didx0], add=True)

        @pl.when(have_b)
        def _():
            base1 = (ch + NC * NS) * K
            pltpu.sync_copy(dst_hbm.at[pl.ds(base1, K)], didx1)
            pltpu.make_async_copy(g_hbm.at[sidx1], rows1, gsem1).wait()
            pltpu.sync_copy(rows1, acc.at[didx1], add=True)

    plsc.subcore_barrier()

    @pl.when(s < NS - 1)
    def _():
        pltpu.sync_copy(acc.at[pl.ds(row0, ROWS_MAIN)],
                        out_hbm.at[c, pl.ds(row0, ROWS_MAIN)])

    @pl.when(s == NS - 1)
    def _():
        pltpu.sync_copy(acc.at[pl.ds((NS - 1) * ROWS_MAIN, ROWS_LAST)],
                        out_hbm.at[c, pl.ds((NS - 1) * ROWS_MAIN, ROWS_LAST)])


def _tc_pre_body(d0_ref, d1_ref, h_ref, norm_ref, g_ref):
    deg = jnp.maximum(d0_ref[...] + d1_ref[...], 1.0)
    norm = lax.rsqrt(deg)
    norm_ref[...] = norm
    g_ref[...] = h_ref[...] * norm


def _tc_pre(d0, d1, h):
    return pl.pallas_call(
        _tc_pre_body,
        out_shape=[
            jax.ShapeDtypeStruct((N, 1), jnp.float32),
            jax.ShapeDtypeStruct((N, D), jnp.float32),
        ],
    )(d0, d1, h)


def _batchnorm_relu(x, gamma, beta):
    mean = jnp.mean(x, axis=0, keepdims=True)
    xc = x - mean
    var = jnp.mean(xc * xc, axis=0, keepdims=True)
    return jnp.maximum(xc * lax.rsqrt(var + EPS) * gamma + beta, 0.0)


def _tc_layer_body(a0_ref, a1_ref, norm_ref, gamma_ref, beta_ref, g_ref):
    x = (a0_ref[...] + a1_ref[...]) * norm_ref[...]
    y = _batchnorm_relu(x, gamma_ref[...], beta_ref[...])
    g_ref[...] = y * norm_ref[...]


def _tc_layer(a0, a1, norm, gamma, beta):
    return pl.pallas_call(
        _tc_layer_body,
        out_shape=jax.ShapeDtypeStruct((N, D), jnp.float32),
    )(a0, a1, norm, gamma, beta)


def _tc_final_body(a0_ref, a1_ref, norm_ref, gamma_ref, beta_ref,
                   W1_ref, b1_ref, mg_ref, mb_ref, W2_ref, b2_ref, out_ref):
    x = (a0_ref[...] + a1_ref[...]) * norm_ref[...]
    y = _batchnorm_relu(x, gamma_ref[...], beta_ref[...])
    x1 = jnp.dot(y, W1_ref[...], preferred_element_type=jnp.float32) + b1_ref[...]
    m1 = jnp.mean(x1, axis=0, keepdims=True)
    x1c = x1 - m1
    v1 = jnp.mean(x1c * x1c, axis=0, keepdims=True)
    xn = x1c * lax.rsqrt(v1 + EPS) * mg_ref[...] + mb_ref[...]
    out_ref[...] = (jnp.dot(xn, W2_ref[...], preferred_element_type=jnp.float32)
                    + b2_ref[...])


def _tc_final(a0, a1, norm, gamma, beta, W1, b1, mg, mb, W2, b2):
    return pl.pallas_call(
        _tc_final_body,
        out_shape=jax.ShapeDtypeStruct((N, C), jnp.float32),
    )(a0, a1, norm, gamma, beta, W1, b1, mg, mb, W2, b2)


def kernel(h, edge_index, e, bn_gamma, bn_beta, W1, b1,
           mbn_gamma, mbn_beta, W2, b2):
    del e
    src = edge_index[0]
    dst = edge_index[1]

    pad = E_PAD - E
    src1d = jnp.concatenate([src, jnp.zeros((pad,), jnp.int32)])
    dst1d = jnp.concatenate(
        [dst, N + (jnp.arange(pad, dtype=jnp.int32) % K)])
    dst2d = dst1d.reshape(NCHUNK_PAD, K)

    zeros_n = jnp.zeros((NPAD,), jnp.float32)
    zeros_nd = jnp.zeros((N, D), jnp.float32)

    deg2 = _sc_degree(dst2d, zeros_n)
    norm, g = _tc_pre(deg2[:N].reshape(N, 1),
                      deg2[NPAD:NPAD + N].reshape(N, 1), h)

    gamma2 = bn_gamma.reshape(1, D)
    beta2 = bn_beta.reshape(1, D)

    out = None
    for layer in range(L):
        agg = _sc_gather_scatter(g, src1d, dst1d, zeros_nd)
        if layer < L - 1:
            g = _tc_layer(agg[0], agg[1], norm, gamma2, beta2)
        else:
            out = _tc_final(agg[0], agg[1], norm, gamma2, beta2,
                            W1, b1.reshape(1, D),
                            mbn_gamma.reshape(1, D), mbn_beta.reshape(1, D),
                            W2, b2.reshape(1, C))
    return out


# speedup vs baseline: 3.5978x; 1.1389x over previous
import functools

import jax
import jax.numpy as jnp
from jax import lax
from jax.experimental import pallas as pl
from jax.experimental.pallas import tpu as pltpu
from jax.experimental.pallas import tpu_sc as plsc

N = 10000
E = 320000
D = 128
C = 10
L = 4
EPS = 1e-5

NC = 2
NS = 16
K = 128
NCHUNK = E // K
CPT = 80
NCHUNK_PAD = NC * NS * CPT
E_PAD = NCHUNK_PAD * K
N_ACC = N + K
ROWS_MAIN = 624
ROWS_LAST = N - (NS - 1) * ROWS_MAIN
NPAD = 10240

_mesh = plsc.VectorSubcoreMesh(core_axis_name="c", subcore_axis_name="s")


@functools.partial(
    pl.kernel,
    out_type=jax.ShapeDtypeStruct((NC * NPAD,), jnp.float32),
    mesh=_mesh,
    scratch_types=[
        pltpu.VMEM((CPT, K), jnp.int32),
        pltpu.VMEM((K,), jnp.float32),
        pltpu.VMEM_SHARED((NPAD,), jnp.float32),
    ],
)
def _sc_degree(dst_hbm, zeros_hbm, out_hbm, didx_all, ones_v, acc):
    c = lax.axis_index("c")
    s = lax.axis_index("s")
    w = c * NS + s

    @pl.when(s == 0)
    def _():
        pltpu.sync_copy(zeros_hbm, acc)

    for i in range(K // 16):
        ones_v[pl.ds(i * 16, 16)] = jnp.ones((16,), jnp.float32)

    pltpu.sync_copy(dst_hbm.at[pl.ds(pl.multiple_of(w * CPT, 8), CPT)], didx_all)
    plsc.subcore_barrier()

    @pl.loop(0, CPT)
    def _(j):
        pltpu.sync_copy(ones_v, acc.at[didx_all.at[j]], add=True)

    plsc.subcore_barrier()

    @pl.when(s == 0)
    def _():
        pltpu.sync_copy(acc, out_hbm.at[pl.ds(pl.multiple_of(c * NPAD, 8), NPAD)])


@functools.partial(
    pl.kernel,
    out_type=jax.ShapeDtypeStruct((NC, N, D), jnp.float32),
    mesh=_mesh,
    scratch_types=[
        pltpu.VMEM((K,), jnp.int32),
        pltpu.VMEM((K,), jnp.int32),
        pltpu.VMEM((K,), jnp.int32),
        pltpu.VMEM((K,), jnp.int32),
        pltpu.VMEM((K, D), jnp.float32),
        pltpu.VMEM((K, D), jnp.float32),
        pltpu.VMEM_SHARED((N_ACC, D), jnp.float32),
        pltpu.SemaphoreType.DMA,
        pltpu.SemaphoreType.DMA,
    ],
)
def _sc_gather_scatter(g_hbm, src_hbm, dst_hbm, zeros_hbm, out_hbm,
                       sidx0, sidx1, didx0, didx1, rows0, rows1, acc,
                       gsem0, gsem1):
    c = lax.axis_index("c")
    s = lax.axis_index("s")
    w = c * NS + s
    row0 = pl.multiple_of(s * ROWS_MAIN, 8)

    @pl.when(s < NS - 1)
    def _():
        pltpu.sync_copy(zeros_hbm.at[pl.ds(row0, ROWS_MAIN)],
                        acc.at[pl.ds(row0, ROWS_MAIN)])

    @pl.when(s == NS - 1)
    def _():
        pltpu.sync_copy(zeros_hbm.at[pl.ds((NS - 1) * ROWS_MAIN, ROWS_LAST)],
                        acc.at[pl.ds((NS - 1) * ROWS_MAIN, ROWS_LAST)])

    plsc.subcore_barrier()

    cw = c * NS + s
    pltpu.sync_copy(src_hbm.at[pl.ds(cw * K, K)], sidx0)
    pltpu.async_copy(g_hbm.at[sidx0], rows0, gsem0)
    pltpu.sync_copy(src_hbm.at[pl.ds((cw + 32) * K, K)], sidx1)
    pltpu.async_copy(g_hbm.at[sidx1], rows1, gsem1)

    @pl.loop(cw, NCHUNK, step=2 * NC * NS)
    def _(ch):
        pltpu.sync_copy(dst_hbm.at[pl.ds(ch * K, K)], didx0)
        pltpu.make_async_copy(g_hbm.at[sidx0], rows0, gsem0).wait()
        pltpu.sync_copy(rows0, acc.at[didx0], add=True)

        @pl.when(ch + 64 < NCHUNK)
        def _():
            pltpu.sync_copy(src_hbm.at[pl.ds((ch + 64) * K, K)], sidx0)
            pltpu.async_copy(g_hbm.at[sidx0], rows0, gsem0)

        @pl.when(ch + 32 < NCHUNK)
        def _():
            pltpu.sync_copy(dst_hbm.at[pl.ds((ch + 32) * K, K)], didx1)
            pltpu.make_async_copy(g_hbm.at[sidx1], rows1, gsem1).wait()
            pltpu.sync_copy(rows1, acc.at[didx1], add=True)

        @pl.when(ch + 96 < NCHUNK)
        def _():
            pltpu.sync_copy(src_hbm.at[pl.ds((ch + 96) * K, K)], sidx1)
            pltpu.async_copy(g_hbm.at[sidx1], rows1, gsem1)

    plsc.subcore_barrier()

    @pl.when(s < NS - 1)
    def _():
        pltpu.sync_copy(acc.at[pl.ds(row0, ROWS_MAIN)],
                        out_hbm.at[c, pl.ds(row0, ROWS_MAIN)])

    @pl.when(s == NS - 1)
    def _():
        pltpu.sync_copy(acc.at[pl.ds((NS - 1) * ROWS_MAIN, ROWS_LAST)],
                        out_hbm.at[c, pl.ds((NS - 1) * ROWS_MAIN, ROWS_LAST)])


def _tc_pre_body(d0_ref, d1_ref, h_ref, norm_ref, g_ref):
    deg = jnp.maximum(d0_ref[...] + d1_ref[...], 1.0)
    norm = lax.rsqrt(deg)
    norm_ref[...] = norm
    g_ref[...] = h_ref[...] * norm


def _tc_pre(d0, d1, h):
    return pl.pallas_call(
        _tc_pre_body,
        out_shape=[
            jax.ShapeDtypeStruct((N, 1), jnp.float32),
            jax.ShapeDtypeStruct((N, D), jnp.float32),
        ],
    )(d0, d1, h)


def _batchnorm_relu(x, gamma, beta):
    mean = jnp.mean(x, axis=0, keepdims=True)
    xc = x - mean
    var = jnp.mean(xc * xc, axis=0, keepdims=True)
    return jnp.maximum(xc * lax.rsqrt(var + EPS) * gamma + beta, 0.0)


def _tc_layer_body(a0_ref, a1_ref, norm_ref, gamma_ref, beta_ref, g_ref):
    x = (a0_ref[...] + a1_ref[...]) * norm_ref[...]
    y = _batchnorm_relu(x, gamma_ref[...], beta_ref[...])
    g_ref[...] = y * norm_ref[...]


def _tc_layer(a0, a1, norm, gamma, beta):
    return pl.pallas_call(
        _tc_layer_body,
        out_shape=jax.ShapeDtypeStruct((N, D), jnp.float32),
    )(a0, a1, norm, gamma, beta)


def _tc_final_body(a0_ref, a1_ref, norm_ref, gamma_ref, beta_ref,
                   W1_ref, b1_ref, mg_ref, mb_ref, W2_ref, b2_ref, out_ref):
    x = (a0_ref[...] + a1_ref[...]) * norm_ref[...]
    y = _batchnorm_relu(x, gamma_ref[...], beta_ref[...])
    x1 = jnp.dot(y, W1_ref[...], preferred_element_type=jnp.float32) + b1_ref[...]
    m1 = jnp.mean(x1, axis=0, keepdims=True)
    x1c = x1 - m1
    v1 = jnp.mean(x1c * x1c, axis=0, keepdims=True)
    xn = x1c * lax.rsqrt(v1 + EPS) * mg_ref[...] + mb_ref[...]
    out_ref[...] = (jnp.dot(xn, W2_ref[...], preferred_element_type=jnp.float32)
                    + b2_ref[...])


def _tc_final(a0, a1, norm, gamma, beta, W1, b1, mg, mb, W2, b2):
    return pl.pallas_call(
        _tc_final_body,
        out_shape=jax.ShapeDtypeStruct((N, C), jnp.float32),
    )(a0, a1, norm, gamma, beta, W1, b1, mg, mb, W2, b2)


def kernel(h, edge_index, e, bn_gamma, bn_beta, W1, b1,
           mbn_gamma, mbn_beta, W2, b2):
    del e
    src = edge_index[0]
    dst = edge_index[1]

    pad = E_PAD - E
    src1d = jnp.concatenate([src, jnp.zeros((pad,), jnp.int32)])
    dst1d = jnp.concatenate(
        [dst, N + (jnp.arange(pad, dtype=jnp.int32) % K)])
    dst2d = dst1d.reshape(NCHUNK_PAD, K)

    zeros_n = jnp.zeros((NPAD,), jnp.float32)
    zeros_nd = jnp.zeros((N, D), jnp.float32)

    deg2 = _sc_degree(dst2d, zeros_n)
    norm, g = _tc_pre(deg2[:N].reshape(N, 1),
                      deg2[NPAD:NPAD + N].reshape(N, 1), h)

    gamma2 = bn_gamma.reshape(1, D)
    beta2 = bn_beta.reshape(1, D)

    out = None
    for layer in range(L):
        agg = _sc_gather_scatter(g, src1d, dst1d, zeros_nd)
        if layer < L - 1:
            g = _tc_layer(agg[0], agg[1], norm, gamma2, beta2)
        else:
            out = _tc_final(agg[0], agg[1], norm, gamma2, beta2,
                            W1, b1.reshape(1, D),
                            mbn_gamma.reshape(1, D), mbn_beta.reshape(1, D),
                            W2, b2.reshape(1, C))
    return out


# speedup vs baseline: 4.1823x; 1.1625x over previous
import functools

import jax
import jax.numpy as jnp
from jax import lax
from jax.experimental import pallas as pl
from jax.experimental.pallas import tpu as pltpu
from jax.experimental.pallas import tpu_sc as plsc

N = 10000
E = 320000
D = 128
C = 10
L = 4
EPS = 1e-5

NC = 2
NS = 16
K = 128
NCHUNK = E // K
CPT = 80
NCHUNK_PAD = NC * NS * CPT
E_PAD = NCHUNK_PAD * K
N_ACC = N + K
ROWS_MAIN = 624
ROWS_LAST = N - (NS - 1) * ROWS_MAIN
NPAD = 10240

_mesh = plsc.VectorSubcoreMesh(core_axis_name="c", subcore_axis_name="s")


@functools.partial(
    pl.kernel,
    out_type=jax.ShapeDtypeStruct((NC * NPAD,), jnp.float32),
    mesh=_mesh,
    scratch_types=[
        pltpu.VMEM((CPT, K), jnp.int32),
        pltpu.VMEM((K,), jnp.float32),
        pltpu.VMEM_SHARED((NPAD,), jnp.float32),
    ],
)
def _sc_degree(dst_hbm, zeros_hbm, out_hbm, didx_all, ones_v, acc):
    c = lax.axis_index("c")
    s = lax.axis_index("s")
    w = c * NS + s

    @pl.when(s == 0)
    def _():
        pltpu.sync_copy(zeros_hbm, acc)

    for i in range(K // 16):
        ones_v[pl.ds(i * 16, 16)] = jnp.ones((16,), jnp.float32)

    pltpu.sync_copy(dst_hbm.at[pl.ds(pl.multiple_of(w * CPT, 8), CPT)], didx_all)
    plsc.subcore_barrier()

    @pl.loop(0, CPT)
    def _(j):
        pltpu.sync_copy(ones_v, acc.at[didx_all.at[j]], add=True)

    plsc.subcore_barrier()

    @pl.when(s == 0)
    def _():
        pltpu.sync_copy(acc, out_hbm.at[pl.ds(pl.multiple_of(c * NPAD, 8), NPAD)])


@functools.partial(
    pl.kernel,
    out_type=jax.ShapeDtypeStruct((NC, N, D), jnp.float32),
    mesh=_mesh,
    scratch_types=[
        pltpu.VMEM((K,), jnp.int32),
        pltpu.VMEM((K,), jnp.int32),
        pltpu.VMEM((K,), jnp.int32),
        pltpu.VMEM((K,), jnp.int32),
        pltpu.VMEM((K, D), jnp.float32),
        pltpu.VMEM((K, D), jnp.float32),
        pltpu.VMEM_SHARED((N_ACC, D), jnp.float32),
        pltpu.SemaphoreType.DMA,
        pltpu.SemaphoreType.DMA,
        pltpu.SemaphoreType.DMA,
        pltpu.SemaphoreType.DMA,
    ],
)
def _sc_gather_scatter(g_hbm, src_hbm, dst_hbm, zeros_hbm, out_hbm,
                       sidx0, sidx1, didx0, didx1, rows0, rows1, acc,
                       gsem0, gsem1, dsem0, dsem1):
    c = lax.axis_index("c")
    s = lax.axis_index("s")
    w = c * NS + s
    row0 = pl.multiple_of(s * ROWS_MAIN, 8)

    @pl.when(s < NS - 1)
    def _():
        pltpu.sync_copy(zeros_hbm.at[pl.ds(row0, ROWS_MAIN)],
                        acc.at[pl.ds(row0, ROWS_MAIN)])

    @pl.when(s == NS - 1)
    def _():
        pltpu.sync_copy(zeros_hbm.at[pl.ds((NS - 1) * ROWS_MAIN, ROWS_LAST)],
                        acc.at[pl.ds((NS - 1) * ROWS_MAIN, ROWS_LAST)])

    plsc.subcore_barrier()

    cw = c * NS + s
    pltpu.sync_copy(src_hbm.at[pl.ds(cw * K, K)], sidx0)
    pltpu.async_copy(g_hbm.at[sidx0], rows0, gsem0)
    pltpu.async_copy(dst_hbm.at[pl.ds(cw * K, K)], didx0, dsem0)
    pltpu.sync_copy(src_hbm.at[pl.ds((cw + 32) * K, K)], sidx1)
    pltpu.async_copy(g_hbm.at[sidx1], rows1, gsem1)
    pltpu.async_copy(dst_hbm.at[pl.ds((cw + 32) * K, K)], didx1, dsem1)

    @pl.loop(cw, NCHUNK, step=2 * NC * NS)
    def _(ch):
        pltpu.make_async_copy(dst_hbm.at[pl.ds(ch * K, K)], didx0, dsem0).wait()
        pltpu.make_async_copy(g_hbm.at[sidx0], rows0, gsem0).wait()
        pltpu.sync_copy(rows0, acc.at[didx0], add=True)

        @pl.when(ch + 64 < NCHUNK)
        def _():
            pltpu.sync_copy(src_hbm.at[pl.ds((ch + 64) * K, K)], sidx0)
            pltpu.async_copy(g_hbm.at[sidx0], rows0, gsem0)
            pltpu.async_copy(dst_hbm.at[pl.ds((ch + 64) * K, K)], didx0, dsem0)

        @pl.when(ch + 32 < NCHUNK)
        def _():
            pltpu.make_async_copy(dst_hbm.at[pl.ds((ch + 32) * K, K)],
                                  didx1, dsem1).wait()
            pltpu.make_async_copy(g_hbm.at[sidx1], rows1, gsem1).wait()
            pltpu.sync_copy(rows1, acc.at[didx1], add=True)

        @pl.when(ch + 96 < NCHUNK)
        def _():
            pltpu.sync_copy(src_hbm.at[pl.ds((ch + 96) * K, K)], sidx1)
            pltpu.async_copy(g_hbm.at[sidx1], rows1, gsem1)
            pltpu.async_copy(dst_hbm.at[pl.ds((ch + 96) * K, K)], didx1, dsem1)

    plsc.subcore_barrier()

    @pl.when(s < NS - 1)
    def _():
        pltpu.sync_copy(acc.at[pl.ds(row0, ROWS_MAIN)],
                        out_hbm.at[c, pl.ds(row0, ROWS_MAIN)])

    @pl.when(s == NS - 1)
    def _():
        pltpu.sync_copy(acc.at[pl.ds((NS - 1) * ROWS_MAIN, ROWS_LAST)],
                        out_hbm.at[c, pl.ds((NS - 1) * ROWS_MAIN, ROWS_LAST)])


def _tc_pre_body(d0_ref, d1_ref, h_ref, norm_ref, g_ref):
    deg = jnp.maximum(d0_ref[...] + d1_ref[...], 1.0)
    norm = lax.rsqrt(deg)
    norm_ref[...] = norm
    g_ref[...] = h_ref[...] * norm


def _tc_pre(d0, d1, h):
    return pl.pallas_call(
        _tc_pre_body,
        out_shape=[
            jax.ShapeDtypeStruct((N, 1), jnp.float32),
            jax.ShapeDtypeStruct((N, D), jnp.float32),
        ],
    )(d0, d1, h)


def _batchnorm_relu(x, gamma, beta):
    mean = jnp.mean(x, axis=0, keepdims=True)
    xc = x - mean
    var = jnp.mean(xc * xc, axis=0, keepdims=True)
    return jnp.maximum(xc * lax.rsqrt(var + EPS) * gamma + beta, 0.0)


def _tc_layer_body(a0_ref, a1_ref, norm_ref, gamma_ref, beta_ref, g_ref):
    x = (a0_ref[...] + a1_ref[...]) * norm_ref[...]
    y = _batchnorm_relu(x, gamma_ref[...], beta_ref[...])
    g_ref[...] = y * norm_ref[...]


def _tc_layer(a0, a1, norm, gamma, beta):
    return pl.pallas_call(
        _tc_layer_body,
        out_shape=jax.ShapeDtypeStruct((N, D), jnp.float32),
    )(a0, a1, norm, gamma, beta)


def _tc_final_body(a0_ref, a1_ref, norm_ref, gamma_ref, beta_ref,
                   W1_ref, b1_ref, mg_ref, mb_ref, W2_ref, b2_ref, out_ref):
    x = (a0_ref[...] + a1_ref[...]) * norm_ref[...]
    y = _batchnorm_relu(x, gamma_ref[...], beta_ref[...])
    x1 = jnp.dot(y, W1_ref[...], preferred_element_type=jnp.float32) + b1_ref[...]
    m1 = jnp.mean(x1, axis=0, keepdims=True)
    x1c = x1 - m1
    v1 = jnp.mean(x1c * x1c, axis=0, keepdims=True)
    xn = x1c * lax.rsqrt(v1 + EPS) * mg_ref[...] + mb_ref[...]
    out_ref[...] = (jnp.dot(xn, W2_ref[...], preferred_element_type=jnp.float32)
                    + b2_ref[...])


def _tc_final(a0, a1, norm, gamma, beta, W1, b1, mg, mb, W2, b2):
    return pl.pallas_call(
        _tc_final_body,
        out_shape=jax.ShapeDtypeStruct((N, C), jnp.float32),
    )(a0, a1, norm, gamma, beta, W1, b1, mg, mb, W2, b2)


def kernel(h, edge_index, e, bn_gamma, bn_beta, W1, b1,
           mbn_gamma, mbn_beta, W2, b2):
    del e
    src = edge_index[0]
    dst = edge_index[1]

    pad = E_PAD - E
    src1d = jnp.concatenate([src, jnp.zeros((pad,), jnp.int32)])
    dst1d = jnp.concatenate(
        [dst, N + (jnp.arange(pad, dtype=jnp.int32) % K)])
    dst2d = dst1d.reshape(NCHUNK_PAD, K)

    zeros_n = jnp.zeros((NPAD,), jnp.float32)
    zeros_nd = jnp.zeros((N, D), jnp.float32)

    deg2 = _sc_degree(dst2d, zeros_n)
    norm, g = _tc_pre(deg2[:N].reshape(N, 1),
                      deg2[NPAD:NPAD + N].reshape(N, 1), h)

    gamma2 = bn_gamma.reshape(1, D)
    beta2 = bn_beta.reshape(1, D)

    out = None
    for layer in range(L):
        agg = _sc_gather_scatter(g, src1d, dst1d, zeros_nd)
        if layer < L - 1:
            g = _tc_layer(agg[0], agg[1], norm, gamma2, beta2)
        else:
            out = _tc_final(agg[0], agg[1], norm, gamma2, beta2,
                            W1, b1.reshape(1, D),
                            mbn_gamma.reshape(1, D), mbn_beta.reshape(1, D),
                            W2, b2.reshape(1, C))
    return out
